# Initial kernel scaffold; baseline (speedup 1.0000x reference)
import jax
import jax.numpy as jnp
from jax.experimental import pallas as pl


def kernel(x, edge_index, batch, W_l, b_l, W_r, gn_weight, gn_bias, gn_mean_scale, W1, b1, W2, b2):
    raise NotImplementedError("write your pallas kernel here")


# speedup vs baseline: 6.0848x; 6.0848x over previous
import functools

import jax
import jax.numpy as jnp
from jax import lax
from jax.experimental import pallas as pl
from jax.experimental.pallas import tpu as pltpu
from jax.experimental.pallas import tpu_sc as plsc

N = 10000
N_PAD = 10240
E = 320000
D = 128
G = 64

NC = 2
NS = 16
NW = NC * NS
EPT = E // NW
K = 80
ROWS_PER_TILE = N_PAD // NS

B = 1000
NB = N // B


W = 128
NSTAGE = ROWS_PER_TILE // W
NCH = EPT // K
NPH = 5
CPP = NCH // NPH


def _sc_agg(x_hbm, src3_hbm, dst3_hbm, zeros_nd,
            agg_out,
            src_v, dst_v, rows_v,
            agg_sh, sem):
    c = lax.axis_index("c")
    s = lax.axis_index("s")
    wid = s * NC + c
    r0 = s * ROWS_PER_TILE

    def zbody(j, carry):
        pltpu.sync_copy(zeros_nd.at[pl.ds(0, W)],
                        agg_sh.at[pl.ds(r0 + j * W, W)])
        return carry

    lax.fori_loop(0, NSTAGE, zbody, 0)
    plsc.subcore_barrier()

    for p in range(NPH):
        pltpu.sync_copy(src3_hbm.at[wid, p], src_v)
        pltpu.sync_copy(dst3_hbm.at[wid, p], dst_v)
        for i in range(CPP):
            pltpu.async_copy(x_hbm.at[src_v.at[i]], rows_v, sem).wait()
            pltpu.sync_copy(rows_v, agg_sh.at[dst_v.at[i]], add=True)
    plsc.subcore_barrier()

    def wbody(j, carry):
        pltpu.sync_copy(agg_sh.at[pl.ds(r0 + j * W, W)],
                        agg_out.at[c, pl.ds(r0 + j * W, W)])
        return carry

    lax.fori_loop(0, NSTAGE, wbody, 0)


def _sc_cnt(dst3_hbm, zeros_nd, ones_kd,
            cnt_out,
            dst_v, ones_v,
            cnt_sh, sem):
    c = lax.axis_index("c")
    s = lax.axis_index("s")
    wid = s * NC + c
    r0 = s * ROWS_PER_TILE

    def zbody(j, carry):
        pltpu.sync_copy(zeros_nd.at[pl.ds(0, W)],
                        cnt_sh.at[pl.ds(r0 + j * W, W)])
        return carry

    lax.fori_loop(0, NSTAGE, zbody, 0)
    pltpu.sync_copy(ones_kd, ones_v)
    plsc.subcore_barrier()

    for p in range(NPH):
        pltpu.sync_copy(dst3_hbm.at[wid, p], dst_v)
        for i in range(CPP):
            pltpu.sync_copy(ones_v, cnt_sh.at[dst_v.at[i]], add=True)
    plsc.subcore_barrier()

    def wbody(j, carry):
        pltpu.sync_copy(cnt_sh.at[pl.ds(r0 + j * W, W)],
                        cnt_out.at[c, pl.ds(r0 + j * W, W)])
        return carry

    lax.fori_loop(0, NSTAGE, wbody, 0)


@functools.lru_cache(maxsize=1)
def _make_sc_call():
    return functools.partial(
        pl.kernel,
        out_type=jax.ShapeDtypeStruct((NC, N_PAD, D), jnp.float32),
        mesh=plsc.VectorSubcoreMesh(core_axis_name="c", subcore_axis_name="s"),
        scratch_types=[
            pltpu.VMEM((CPP, K), jnp.int32),
            pltpu.VMEM((CPP, K), jnp.int32),
            pltpu.VMEM((K, D), jnp.float32),
            pltpu.VMEM_SHARED((N_PAD, D), jnp.float32),
            pltpu.SemaphoreType.DMA,
        ],
    )(_sc_agg)


@functools.lru_cache(maxsize=1)
def _make_cnt_call():
    return functools.partial(
        pl.kernel,
        out_type=jax.ShapeDtypeStruct((NC, N_PAD, D), jnp.float32),
        mesh=plsc.VectorSubcoreMesh(core_axis_name="c", subcore_axis_name="s"),
        scratch_types=[
            pltpu.VMEM((CPP, K), jnp.int32),
            pltpu.VMEM((K, D), jnp.float32),
            pltpu.VMEM_SHARED((N_PAD, D), jnp.float32),
            pltpu.SemaphoreType.DMA,
        ],
    )(_sc_cnt)


def _tc1(agg_ref, cnt_ref, x_ref, batch_ref, wl_ref, bl_ref, wr_ref,
         w1_ref, b1_ref, w2_ref, b2_ref,
         conv_ref, ax_ref, sums_ref, sumsq_ref, cntg_ref):
    i = pl.program_id(0)
    x = x_ref[...]
    agg = agg_ref[0] + agg_ref[1]
    cnt = cnt_ref[0, :, 0] + cnt_ref[1, :, 0]
    agg = agg / jnp.clip(cnt, 1.0, None)[:, None]
    conv = (lax.dot_general(agg, wl_ref[...], (((1,), (1,)), ((), ())),
                            preferred_element_type=jnp.float32)
            + bl_ref[...]
            + lax.dot_general(x, wr_ref[...], (((1,), (1,)), ((), ())),
                              preferred_element_type=jnp.float32))
    t = jnp.maximum(
        lax.dot_general(x, w1_ref[...], (((1,), (1,)), ((), ())),
                        preferred_element_type=jnp.float32) + b1_ref[...], 0.0)
    gate = jax.nn.sigmoid(
        lax.dot_general(t, w2_ref[...], (((1,), (1,)), ((), ())),
                        preferred_element_type=jnp.float32) + b2_ref[...])
    ax = gate * x
    conv_ref[...] = conv
    ax_ref[...] = ax

    bvec = batch_ref[0]
    onehot_t = (jnp.broadcast_to(bvec, (G, B))
                == lax.broadcasted_iota(jnp.int32, (G, B), 0)).astype(jnp.float32)
    s0 = lax.dot_general(onehot_t, conv, (((1,), (0,)), ((), ())),
                         preferred_element_type=jnp.float32)
    s1 = lax.dot_general(onehot_t, conv * conv, (((1,), (0,)), ((), ())),
                         preferred_element_type=jnp.float32)
    s2 = jnp.broadcast_to(jnp.sum(onehot_t, axis=1)[:, None], (G, D))

    @pl.when(i == 0)
    def _():
        sums_ref[...] = s0
        sumsq_ref[...] = s1
        cntg_ref[...] = s2

    @pl.when(i > 0)
    def _():
        sums_ref[...] += s0
        sumsq_ref[...] += s1
        cntg_ref[...] += s2


def _tc2(conv_ref, ax_ref, batch_ref, sums_ref, sumsq_ref, cntg_ref,
         gw_ref, gb_ref, gs_ref, out_ref):
    cg = jnp.clip(cntg_ref[...], 1.0, None)
    mean = sums_ref[...] / cg
    msq = sumsq_ref[...] / cg
    sca = gs_ref[...]
    var = msq - (2.0 * sca - sca * sca) * mean * mean
    inv = 1.0 / jnp.sqrt(var + 1e-5)
    wrow = gw_ref[...] * inv
    mrow = sca * mean
    bvec = batch_ref[0]
    onehot_t = (jnp.broadcast_to(bvec, (G, B))
                == lax.broadcasted_iota(jnp.int32, (G, B), 0)).astype(jnp.float32)
    mrows = lax.dot_general(onehot_t, mrow, (((0,), (0,)), ((), ())),
                            preferred_element_type=jnp.float32)
    wrows = lax.dot_general(onehot_t, wrow, (((0,), (0,)), ((), ())),
                            preferred_element_type=jnp.float32)
    out = jnp.maximum((conv_ref[...] - mrows) * wrows + gb_ref[...]
                      + ax_ref[...], 0.0)
    out_ref[...] = out


_full = lambda shape: pl.BlockSpec(shape, lambda i: tuple(0 for _ in shape))

_TC1_IN_SPECS = [
    pl.BlockSpec((NC, B, D), lambda i: (0, i, 0)),
    pl.BlockSpec((NC, B, D), lambda i: (0, i, 0)),
    pl.BlockSpec((B, D), lambda i: (i, 0)),
    pl.BlockSpec((1, 1, B), lambda i: (i, 0, 0)),
    _full((D, D)),
    _full((1, D)),
    _full((D, D)),
    _full((16, D)),
    _full((1, 16)),
    _full((D, 16)),
    _full((1, D)),
]
_TC1_OUT_SPECS = [
    pl.BlockSpec((B, D), lambda i: (i, 0)),
    pl.BlockSpec((B, D), lambda i: (i, 0)),
    _full((G, D)),
    _full((G, D)),
    _full((G, D)),
]
_TC1_OUT_SHAPE = [
    jax.ShapeDtypeStruct((N, D), jnp.float32),
    jax.ShapeDtypeStruct((N, D), jnp.float32),
    jax.ShapeDtypeStruct((G, D), jnp.float32),
    jax.ShapeDtypeStruct((G, D), jnp.float32),
    jax.ShapeDtypeStruct((G, D), jnp.float32),
]

_tc1_call = pl.pallas_call(
    _tc1,
    grid=(NB,),
    in_specs=_TC1_IN_SPECS,
    out_specs=_TC1_OUT_SPECS,
    out_shape=_TC1_OUT_SHAPE,
)

_TC2_IN_SPECS = [
    pl.BlockSpec((B, D), lambda i: (i, 0)),
    pl.BlockSpec((B, D), lambda i: (i, 0)),
    pl.BlockSpec((1, 1, B), lambda i: (i, 0, 0)),
    _full((G, D)),
    _full((G, D)),
    _full((G, D)),
    _full((1, D)),
    _full((1, D)),
    _full((1, D)),
]
_TC2_OUT_SPECS = pl.BlockSpec((B, D), lambda i: (i, 0))
_TC2_OUT_SHAPE = jax.ShapeDtypeStruct((N, D), jnp.float32)

_tc2_call = pl.pallas_call(
    _tc2,
    grid=(NB,),
    in_specs=_TC2_IN_SPECS,
    out_specs=_TC2_OUT_SPECS,
    out_shape=_TC2_OUT_SHAPE,
)


def kernel(x, edge_index, batch, W_l, b_l, W_r, gn_weight, gn_bias,
           gn_mean_scale, W1, b1, W2, b2):
    x = x.astype(jnp.float32)
    src = edge_index[0].astype(jnp.int32)
    dst = edge_index[1].astype(jnp.int32)
    src3 = src.reshape(NW, NPH, CPP, K)
    dst3 = dst.reshape(NW, NPH, CPP, K)
    batch3 = batch.astype(jnp.int32).reshape(NB, 1, B)

    zeros_nd = jnp.zeros((W, D), jnp.float32)
    ones_kd = jnp.ones((K, D), jnp.float32)

    agg_parts = _make_sc_call()(x, src3, dst3, zeros_nd)
    cnt_parts = _make_cnt_call()(dst3, zeros_nd, ones_kd)
    agg_parts = agg_parts[:, :N]
    cnt_parts = cnt_parts[:, :N]

    conv, ax, sums, sumsq, cntg = _tc1_call(
        agg_parts, cnt_parts, x, batch3,
        W_l, b_l.reshape(1, D), W_r,
        W1, b1.reshape(1, 16), W2, b2.reshape(1, D))

    out = _tc2_call(conv, ax, batch3, sums, sumsq, cntg,
                    gn_weight.reshape(1, D), gn_bias.reshape(1, D),
                    gn_mean_scale.reshape(1, D))
    return out
